# baseline (speedup 1.0000x reference)
import jax
import jax.numpy as jnp
from jax.experimental import pallas as pl
from jax.experimental.pallas import tpu as pltpu


def _row_sum(mask_v, b, T):
    return jnp.sum(mask_v[b, :])


def _body(B, T, mask_any, hidden_ref, out_ref, mask_v, m_sem, g_sem):
    m_copies = []
    for b in range(B):
        m = pltpu.make_async_copy(
            mask_any.at[pl.ds(b, 1), :], mask_v.at[pl.ds(b, 1), :], m_sem
        )
        m.start()
        m_copies.append(m)
    spec = []
    for b in range(B):
        c = pltpu.make_async_copy(
            hidden_ref.at[b, pl.ds(T - 1, 1), :],
            out_ref.at[pl.ds(b, 1), :],
            g_sem,
        )
        c.start()
        spec.append(c)
    lasts = []
    for b in range(B):
        m_copies[b].wait()
        lasts.append(jnp.maximum(_row_sum(mask_v, b, T) - 1, 0))
    for c in spec:
        c.wait()
    mispredicted = lasts[0] != T - 1
    for b in range(1, B):
        mispredicted = mispredicted | (lasts[b] != T - 1)

    @pl.when(mispredicted)
    def _():
        for b in range(B):
            @pl.when(lasts[b] != T - 1)
            def _(b=b):
                fix = pltpu.make_async_copy(
                    hidden_ref.at[b, pl.ds(lasts[b], 1), :],
                    out_ref.at[pl.ds(b, 1), :],
                    g_sem,
                )
                fix.start()
                fix.wait()


def kernel(last_hidden_state, attention_mask):
    B, T, H = last_hidden_state.shape
    mask = attention_mask.astype(jnp.int32)
    return pl.pallas_call(
        lambda *refs: _body(B, T, *refs),
        out_shape=jax.ShapeDtypeStruct((B, H), jnp.float32),
        in_specs=[
            pl.BlockSpec(memory_space=pl.ANY),
            pl.BlockSpec(memory_space=pl.ANY),
        ],
        out_specs=pl.BlockSpec(memory_space=pltpu.VMEM),
        scratch_shapes=[
            pltpu.VMEM((B, T), jnp.int32),
            pltpu.SemaphoreType.DMA,
            pltpu.SemaphoreType.DMA,
        ],
    )(mask, last_hidden_state)


# speedup vs baseline: 1.1346x; 1.1346x over previous
import jax
import jax.numpy as jnp
from jax.experimental import pallas as pl
from jax.experimental.pallas import tpu as pltpu


def _body(B, T, mask_any, hidden_ref, out_ref, mask_v, m_sem, g_sem):
    m_copy = pltpu.make_async_copy(mask_any, mask_v, m_sem)
    m_copy.start()
    spec = []
    for b in range(B):
        c = pltpu.make_async_copy(
            hidden_ref.at[b, pl.ds(T - 1, 1), :],
            out_ref.at[pl.ds(b, 1), :],
            g_sem,
        )
        c.start()
        spec.append(c)
    m_copy.wait()
    lasts = [jnp.maximum(jnp.sum(mask_v[b, :]) - 1, 0) for b in range(B)]
    for c in spec:
        c.wait()
    mispredicted = lasts[0] != T - 1
    for b in range(1, B):
        mispredicted = mispredicted | (lasts[b] != T - 1)

    @pl.when(mispredicted)
    def _():
        for b in range(B):
            @pl.when(lasts[b] != T - 1)
            def _(b=b):
                fix = pltpu.make_async_copy(
                    hidden_ref.at[b, pl.ds(lasts[b], 1), :],
                    out_ref.at[pl.ds(b, 1), :],
                    g_sem,
                )
                fix.start()
                fix.wait()


def kernel(last_hidden_state, attention_mask):
    B, T, H = last_hidden_state.shape
    mask = attention_mask.astype(jnp.int32)
    return pl.pallas_call(
        lambda *refs: _body(B, T, *refs),
        out_shape=jax.ShapeDtypeStruct((B, H), jnp.float32),
        in_specs=[
            pl.BlockSpec(memory_space=pl.ANY),
            pl.BlockSpec(memory_space=pl.ANY),
        ],
        out_specs=pl.BlockSpec(memory_space=pltpu.VMEM),
        scratch_shapes=[
            pltpu.VMEM((B, T), jnp.int32),
            pltpu.SemaphoreType.DMA,
            pltpu.SemaphoreType.DMA,
        ],
    )(mask, last_hidden_state)
